# Initial kernel scaffold; baseline (speedup 1.0000x reference)
import jax
import jax.numpy as jnp
from jax.experimental import pallas as pl
from jax.experimental.pallas import tpu as pltpu

N = 512
F = 256
TAU = 0.1
EPS = 1e-10
BI = 8
NBLK = N // BI
JC = 256
NJC = N // JC
NEG_SLOPE = 0.01
INV_CNT = 1.0 / (N - 1)


def _lrelu(x):
    return jnp.where(x > 0, x, NEG_SLOPE * x)


def _stage1_kernel(x_ref, wp_ref, bp_ref, w1a_ref, w1b_ref, b1_ref, w2_ref,
                   mean_ref, s2_ref, sA, sB):
    i = pl.program_id(0)

    @pl.when(i == 0)
    def _prologue():
        h = jnp.dot(x_ref[:], wp_ref[:], preferred_element_type=jnp.float32) + bp_ref[:]
        sA[:] = jnp.dot(h, w1a_ref[:], preferred_element_type=jnp.float32)
        sB[:] = jnp.dot(h, w1b_ref[:], preferred_element_type=jnp.float32) + b1_ref[:]

    a_blk = sA[pl.ds(i * BI, BI), :]
    w2 = w2_ref[:]
    seg = jnp.zeros((BI, F), jnp.float32)
    for jc in range(NJC):
        b_chunk = sB[pl.ds(jc * JC, JC), :]
        t = _lrelu(a_blk[:, None, :] + b_chunk[None, :, :])
        seg = seg + t.sum(axis=1)
        s2_ref[:, jc * JC:(jc + 1) * JC] = (t * w2[None, :, :]).sum(axis=-1)
    diag = _lrelu(a_blk + sB[pl.ds(i * BI, BI), :])
    mean_ref[:] = (seg - diag) * INV_CNT


def _stage2_kernel(mean_ref, we_ref, be_ref, w2a_ref, w2b_ref, b2_ref,
                   w1_ref, fcob_ref, s2_ref, noise_ref,
                   rel_ref, kidx_ref, kl_ref, sC, sD):
    i = pl.program_id(0)

    @pl.when(i == 0)
    def _prologue():
        h2 = _lrelu(jnp.dot(mean_ref[:], we_ref[:],
                            preferred_element_type=jnp.float32) + be_ref[:])
        sC[:] = jnp.dot(h2, w2a_ref[:], preferred_element_type=jnp.float32)
        sD[:] = jnp.dot(h2, w2b_ref[:], preferred_element_type=jnp.float32) + b2_ref[:]

    c_blk = sC[pl.ds(i * BI, BI), :]
    w1 = w1_ref[:]
    chunks = []
    for jc in range(NJC):
        d_chunk = sD[pl.ds(jc * JC, JC), :]
        t = _lrelu(c_blk[:, None, :] + d_chunk[None, :, :])
        chunks.append((t * w1[None, :, :]).sum(axis=-1))
    s1 = jnp.concatenate(chunks, axis=1)

    logits = s1 + s2_ref[:] + fcob_ref[0, 0]
    col = jax.lax.broadcasted_iota(jnp.int32, (BI, N), 1)
    row = i * BI + jax.lax.broadcasted_iota(jnp.int32, (BI, N), 0)
    neg_inf = jnp.float32(-jnp.inf)
    logits = jnp.where(col == row, neg_inf, logits)

    g = logits + noise_ref[:]
    gmax = g.max(axis=1, keepdims=True)
    idx = jnp.where(g == gmax, col, N)
    k = idx.min(axis=1)
    rel_ref[:] = (col == k[:, None]).astype(jnp.float32)
    kidx_ref[0, 0, :] = k

    m = logits.max(axis=1, keepdims=True)
    e = jnp.exp(logits - m)
    p = e / e.sum(axis=1, keepdims=True)
    ent = -(p * jnp.log(p + 1e-16)).sum()
    acc = jnp.where(i == 0, 0.0, kl_ref[0, 0]) + ent
    kl_ref[0, 0] = jnp.where(i == NBLK - 1, acc * (1.0 / (N * N)), acc)


def kernel(feature_emb, proj_W, proj_b, n2e_W, n2e_b, e2n_W, e2n_b,
           n2e2_W, n2e2_b, fco_W, fco_b):
    f32 = jnp.float32
    bp = proj_b.reshape(1, F)
    w1a, w1b = n2e_W[:F], n2e_W[F:]
    b1 = n2e_b.reshape(1, F)
    be = e2n_b.reshape(1, F)
    w2a, w2b = n2e2_W[:F], n2e2_W[F:]
    b2 = n2e2_b.reshape(1, F)
    wfc1 = fco_W[:F, 0].reshape(1, F)
    wfc2 = fco_W[F:, 0].reshape(1, F)
    fcob = fco_b.reshape(1, 1)

    U = jax.random.uniform(jax.random.key(42), (N, N), dtype=f32)
    noise = -jnp.log(EPS - jnp.log(U + EPS))

    full = lambda shape: pl.BlockSpec(shape, lambda i: (0,) * len(shape))
    rowblk = lambda w: pl.BlockSpec((BI, w), lambda i: (i, 0))

    mean, s2 = pl.pallas_call(
        _stage1_kernel,
        grid=(NBLK,),
        in_specs=[full((N, F)), full((F, F)), full((1, F)), full((F, F)),
                  full((F, F)), full((1, F)), full((1, F))],
        out_specs=[rowblk(F), rowblk(N)],
        out_shape=[jax.ShapeDtypeStruct((N, F), f32),
                   jax.ShapeDtypeStruct((N, N), f32)],
        scratch_shapes=[pltpu.VMEM((N, F), f32), pltpu.VMEM((N, F), f32)],
    )(feature_emb, proj_W, bp, w1a, w1b, b1, wfc2)

    rel, kidx, kl = pl.pallas_call(
        _stage2_kernel,
        grid=(NBLK,),
        in_specs=[full((N, F)), full((F, F)), full((1, F)), full((F, F)),
                  full((F, F)), full((1, F)), full((1, F)), full((1, 1)),
                  rowblk(N), rowblk(N)],
        out_specs=[rowblk(N),
                   pl.BlockSpec((1, 1, BI), lambda i: (i, 0, 0)),
                   pl.BlockSpec((1, 1), lambda i: (0, 0))],
        out_shape=[jax.ShapeDtypeStruct((N, N), f32),
                   jax.ShapeDtypeStruct((NBLK, 1, BI), jnp.int32),
                   jax.ShapeDtypeStruct((1, 1), f32)],
        scratch_shapes=[pltpu.VMEM((N, F), f32), pltpu.VMEM((N, F), f32)],
    )(mean, e2n_W, be, w2a, w2b, b2, wfc1, fcob, s2, noise)

    relation_index = jnp.stack([jnp.arange(N, dtype=jnp.int32),
                                kidx.reshape(N)])
    return rel, relation_index, kl.reshape(())


# speedup vs baseline: 28.3541x; 28.3541x over previous
import jax
import jax.numpy as jnp
from jax.experimental import pallas as pl
from jax.experimental.pallas import tpu as pltpu

N = 512
F = 256
TAU = 0.1
EPS = 1e-10
BI = 8
NBLK = N // BI
JC = 256
NJC = N // JC
NEG_SLOPE = 0.01
INV_CNT = 1.0 / (N - 1)


def _lrelu(x):
    return jnp.where(x > 0, x, NEG_SLOPE * x)


def _stage1_kernel(x_ref, wp_ref, bp_ref, w1a_ref, w1b_ref, b1_ref, w2_ref,
                   mean_ref, s2_ref, sA, sB):
    i = pl.program_id(0)

    @pl.when(i == 0)
    def _prologue():
        h = jnp.dot(x_ref[:], wp_ref[:], preferred_element_type=jnp.float32) + bp_ref[:]
        sA[:] = jnp.dot(h, w1a_ref[:], preferred_element_type=jnp.float32)
        sB[:] = jnp.dot(h, w1b_ref[:], preferred_element_type=jnp.float32) + b1_ref[:]

    a_blk = sA[pl.ds(i * BI, BI), :]
    w2 = w2_ref[:]
    seg = jnp.zeros((BI, F), jnp.float32)
    for jc in range(NJC):
        b_chunk = sB[pl.ds(jc * JC, JC), :]
        t = _lrelu(a_blk[:, None, :] + b_chunk[None, :, :])
        seg = seg + t.sum(axis=1)
        s2_ref[:, jc * JC:(jc + 1) * JC] = (t * w2[None, :, :]).sum(axis=-1)
    diag = _lrelu(a_blk + sB[pl.ds(i * BI, BI), :])
    mean_ref[:] = (seg - diag) * INV_CNT


def _stage2_kernel(mean_ref, we_ref, be_ref, w2a_ref, w2b_ref, b2_ref,
                   w1_ref, fcob_ref, s2_ref, noise_ref,
                   rel_ref, kidx_ref, kl_ref, sC, sD):
    i = pl.program_id(0)

    @pl.when(i == 0)
    def _prologue():
        h2 = _lrelu(jnp.dot(mean_ref[:], we_ref[:],
                            preferred_element_type=jnp.float32) + be_ref[:])
        sC[:] = jnp.dot(h2, w2a_ref[:], preferred_element_type=jnp.float32)
        sD[:] = jnp.dot(h2, w2b_ref[:], preferred_element_type=jnp.float32) + b2_ref[:]

    c_blk = sC[pl.ds(i * BI, BI), :]
    w1 = w1_ref[:]
    chunks = []
    for jc in range(NJC):
        d_chunk = sD[pl.ds(jc * JC, JC), :]
        t = _lrelu(c_blk[:, None, :] + d_chunk[None, :, :])
        chunks.append((t * w1[None, :, :]).sum(axis=-1))
    s1 = jnp.concatenate(chunks, axis=1)

    logits = s1 + s2_ref[:] + fcob_ref[:]
    col = jax.lax.broadcasted_iota(jnp.int32, (BI, N), 1)
    row = i * BI + jax.lax.broadcasted_iota(jnp.int32, (BI, N), 0)
    neg_inf = jnp.float32(-jnp.inf)
    logits = jnp.where(col == row, neg_inf, logits)

    g = logits + noise_ref[:]
    gmax = g.max(axis=1, keepdims=True)
    idx = jnp.where(g == gmax, col, N)
    k = idx.min(axis=1)
    rel_ref[:] = (col == k[:, None]).astype(jnp.float32)
    kidx_ref[0, 0, :] = k

    m = logits.max(axis=1, keepdims=True)
    e = jnp.exp(logits - m)
    p = e / e.sum(axis=1, keepdims=True)
    ent = -(p * jnp.log(p + 1e-16)).sum()
    prev = kl_ref[:]
    acc = jnp.where(i == 0, jnp.zeros_like(prev), prev) + ent
    kl_ref[:] = jnp.where(i == NBLK - 1, acc * (1.0 / (N * N)), acc)


def kernel(feature_emb, proj_W, proj_b, n2e_W, n2e_b, e2n_W, e2n_b,
           n2e2_W, n2e2_b, fco_W, fco_b):
    f32 = jnp.float32
    bp = proj_b.reshape(1, F)
    w1a, w1b = n2e_W[:F], n2e_W[F:]
    b1 = n2e_b.reshape(1, F)
    be = e2n_b.reshape(1, F)
    w2a, w2b = n2e2_W[:F], n2e2_W[F:]
    b2 = n2e2_b.reshape(1, F)
    wfc1 = fco_W[:F, 0].reshape(1, F)
    wfc2 = fco_W[F:, 0].reshape(1, F)
    fcob = fco_b.reshape(1, 1)

    U = jax.random.uniform(jax.random.key(42), (N, N), dtype=f32)
    noise = -jnp.log(EPS - jnp.log(U + EPS))

    full = lambda shape: pl.BlockSpec(shape, lambda i: (0,) * len(shape))
    rowblk = lambda w: pl.BlockSpec((BI, w), lambda i: (i, 0))

    mean, s2 = pl.pallas_call(
        _stage1_kernel,
        grid=(NBLK,),
        in_specs=[full((N, F)), full((F, F)), full((1, F)), full((F, F)),
                  full((F, F)), full((1, F)), full((1, F))],
        out_specs=[rowblk(F), rowblk(N)],
        out_shape=[jax.ShapeDtypeStruct((N, F), f32),
                   jax.ShapeDtypeStruct((N, N), f32)],
        scratch_shapes=[pltpu.VMEM((N, F), f32), pltpu.VMEM((N, F), f32)],
    )(feature_emb, proj_W, bp, w1a, w1b, b1, wfc2)

    rel, kidx, kl = pl.pallas_call(
        _stage2_kernel,
        grid=(NBLK,),
        in_specs=[full((N, F)), full((F, F)), full((1, F)), full((F, F)),
                  full((F, F)), full((1, F)), full((1, F)), full((1, 1)),
                  rowblk(N), rowblk(N)],
        out_specs=[rowblk(N),
                   pl.BlockSpec((1, 1, BI), lambda i: (i, 0, 0)),
                   pl.BlockSpec((1, 1), lambda i: (0, 0))],
        out_shape=[jax.ShapeDtypeStruct((N, N), f32),
                   jax.ShapeDtypeStruct((NBLK, 1, BI), jnp.int32),
                   jax.ShapeDtypeStruct((1, 1), f32)],
        scratch_shapes=[pltpu.VMEM((N, F), f32), pltpu.VMEM((N, F), f32)],
    )(mean, e2n_W, be, w2a, w2b, b2, wfc1, fcob, s2, noise)

    relation_index = jnp.stack([jnp.arange(N, dtype=jnp.int32),
                                kidx.reshape(N)])
    return rel, relation_index, kl.reshape(())
